# baseline (speedup 1.0000x reference)
import functools

import jax
import jax.numpy as jnp
from jax import lax
from jax.experimental import pallas as pl
from jax.experimental.pallas import tpu as pltpu
from jax.experimental.pallas import tpu_sc as plsc

NC = 2
NS = 16
NW = NC * NS
L = 16
WIN = 512
NF = 32


def _scan_table(tabT, sids, gbmin, gbmax, cbuf, wbuf0, wbuf1, wbuf2,
                sem0, sem1, sem2, out, base, bpw):
    n_users = tabT.shape[1]
    lasta = ((n_users - WIN) // 128) * 128
    ngrp = bpw // L

    i16 = lax.iota(jnp.int32, L)
    for h in range(ngrp // L):
        gbmin[pl.ds(h * L, L)] = plsc.load_gather(
            sids, [i16 * L + (h * L * L)])
        gbmax[pl.ds(h * L, L)] = plsc.load_gather(
            sids, [i16 * L + (h * L * L + L - 1)])

    first = sids[pl.ds(0, L)][0]
    last = sids[pl.ds(bpw - L, L)][L - 1]
    wlo0 = jnp.minimum((first >> 7) << 7, lasta)
    nwin = (((last >> 7) << 7) - wlo0) // WIN + 1
    nwin3 = ((nwin + 2) // 3) * 3

    def wstart(k):
        w = jnp.minimum(wlo0 + k * WIN, lasta)
        return pl.multiple_of(w, 128)

    def fire(k, buf, sem):
        pltpu.async_copy(tabT.at[:, pl.ds(wstart(k), WIN)], buf, sem)

    def drain(buf, sem):
        pltpu.make_async_copy(tabT.at[:, pl.ds(0, WIN)], buf, sem).wait()

    def extract(k, buf):
        wlo = wstart(k)
        whi = wlo + WIN
        gl = jnp.int32(0)
        gh = jnp.int32(0)
        for h in range(ngrp // L):
            mx = gbmax[pl.ds(h * L, L)]
            mn = gbmin[pl.ds(h * L, L)]
            gl = gl + plsc.all_reduce_population_count(mx < wlo)[0]
            gh = gh + plsc.all_reduce_population_count(mn < whi)[0]

        @pl.loop(gl, gh)
        def _(g):
            gids = sids[pl.ds(g * L, L)]
            rel = gids - wlo
            m = jnp.logical_and(gids >= wlo, gids < whi)
            rows = i16 + g * L
            for f in range(NF):
                fvec = jnp.full((L,), f, jnp.int32)
                vals = plsc.load_gather(buf, [fvec, rel], mask=m)
                plsc.store_scatter(cbuf, [rows, fvec], vals, mask=m)

    fire(0, wbuf0, sem0)
    fire(1, wbuf1, sem1)
    fire(2, wbuf2, sem2)

    @pl.loop(0, nwin3, step=3)
    def _(j):
        drain(wbuf0, sem0)
        extract(j, wbuf0)
        fire(j + 3, wbuf0, sem0)
        drain(wbuf1, sem1)
        extract(j + 1, wbuf1)
        fire(j + 4, wbuf1, sem1)
        drain(wbuf2, sem2)
        extract(j + 2, wbuf2)
        fire(j + 5, wbuf2, sem2)

    drain(wbuf0, sem0)
    drain(wbuf1, sem1)
    drain(wbuf2, sem2)

    pltpu.sync_copy(cbuf, out.at[pl.ds(base, bpw)])


def _scan_body(bpw, su, si, utabT, itabT, uout, iout,
               sids, gbmin, gbmax, cbuf, wbuf0, wbuf1, wbuf2,
               sem0, sem1, sem2):
    wid = lax.axis_index("s") * NC + lax.axis_index("c")
    base = wid * bpw
    pltpu.sync_copy(su.at[pl.ds(base, bpw)], sids)
    _scan_table(utabT, sids, gbmin, gbmax, cbuf, wbuf0, wbuf1, wbuf2,
                sem0, sem1, sem2, uout, base, bpw)
    pltpu.sync_copy(si.at[pl.ds(base, bpw)], sids)
    _scan_table(itabT, sids, gbmin, gbmax, cbuf, wbuf0, wbuf1, wbuf2,
                sem0, sem1, sem2, iout, base, bpw)


def _sc_scan(su, si, utabT, itabT):
    B = su.shape[0]
    bpw = B // NW
    body = functools.partial(_scan_body, bpw)
    out2 = jax.ShapeDtypeStruct((B, 128), jnp.float32)
    mesh = plsc.VectorSubcoreMesh(
        core_axis_name="c", subcore_axis_name="s", num_cores=NC, num_subcores=NS
    )
    k = pl.kernel(
        body,
        out_type=(out2, out2),
        mesh=mesh,
        compiler_params=pltpu.CompilerParams(needs_layout_passes=False),
        scratch_types=[
            pltpu.VMEM((bpw,), jnp.int32),
            pltpu.VMEM((bpw // L,), jnp.int32),
            pltpu.VMEM((bpw // L,), jnp.int32),
            pltpu.VMEM((bpw, 128), jnp.float32),
            pltpu.VMEM((NF, WIN), jnp.float32),
            pltpu.VMEM((NF, WIN), jnp.float32),
            pltpu.VMEM((NF, WIN), jnp.float32),
            pltpu.SemaphoreType.DMA,
            pltpu.SemaphoreType.DMA,
            pltpu.SemaphoreType.DMA,
        ],
    )
    return k(su, si, utabT, itabT)


def _unperm_body(bpw, semu, semi, invu, invi, uout, iout,
                 uidx, iidx, ub0, ub1, ib0, ib1, sem0, sem1):
    wid = lax.axis_index("s") * NC + lax.axis_index("c")
    nchunk = bpw // 128
    pltpu.sync_copy(invu.at[pl.ds(wid * nchunk, nchunk)], uidx)
    pltpu.sync_copy(invi.at[pl.ds(wid * nchunk, nchunk)], iidx)

    def fire(j, ub, ib, sem):
        pltpu.async_copy(semu.at[uidx.at[j]], ub, sem)
        pltpu.async_copy(semi.at[iidx.at[j]], ib, sem)

    def drain(ub, ib, sem):
        pltpu.make_async_copy(semu.at[uidx.at[0]], ub, sem).wait()
        pltpu.make_async_copy(semi.at[iidx.at[0]], ib, sem).wait()

    fire(0, ub0, ib0, sem0)
    fire(1, ub1, ib1, sem1)
    for j in range(nchunk):
        ub, ib, sem = (ub0, ib0, sem0) if j % 2 == 0 else (ub1, ib1, sem1)
        drain(ub, ib, sem)
        row = wid * bpw + j * 128
        pltpu.sync_copy(ub, uout.at[pl.ds(row, 128)])
        pltpu.sync_copy(ib, iout.at[pl.ds(row, 128)])
        if j + 2 < nchunk:
            fire(j + 2, ub, ib, sem)


def _sc_unpermute(semb_u, semb_i, inv_pu, inv_pi):
    B = semb_u.shape[0]
    bpw = B // NW
    nchunk = bpw // 128
    invu2 = inv_pu.reshape(B // 128, 128)
    invi2 = inv_pi.reshape(B // 128, 128)
    body = functools.partial(_unperm_body, bpw)
    out2 = jax.ShapeDtypeStruct((B, 128), jnp.float32)
    mesh = plsc.VectorSubcoreMesh(
        core_axis_name="c", subcore_axis_name="s", num_cores=NC, num_subcores=NS
    )
    k = pl.kernel(
        body,
        out_type=(out2, out2),
        mesh=mesh,
        scratch_types=[
            pltpu.VMEM((nchunk, 128), jnp.int32),
            pltpu.VMEM((nchunk, 128), jnp.int32),
            pltpu.VMEM((128, 128), jnp.float32),
            pltpu.VMEM((128, 128), jnp.float32),
            pltpu.VMEM((128, 128), jnp.float32),
            pltpu.VMEM((128, 128), jnp.float32),
            pltpu.SemaphoreType.DMA,
            pltpu.SemaphoreType.DMA,
        ],
    )
    return k(semb_u, semb_i, invu2, invi2)


def _mlp_body(utailo, itailo, u_ref, i_ref, uid_ref, iid_ref, tu_ref, ti_ref,
              w1u_ref, w1i_ref, b1_ref, w2_ref, b2_ref,
              w3_ref, b3_ref, o_ref):
    bt = u_ref.shape[0]

    def fix(x, ids2, tail_ref, tailo):
        n = tail_ref.shape[0]
        idb = lax.broadcast_in_dim(ids2, (bt, n), (0, 1))
        rel = jnp.clip(idb - tailo, 0, n - 1)
        oh = (rel == lax.broadcasted_iota(jnp.int32, (bt, n), 1))
        tv = jnp.dot(oh.astype(jnp.float32), tail_ref[...],
                     preferred_element_type=jnp.float32)
        keep = lax.broadcast_in_dim(ids2 < tailo, (bt, x.shape[1]), (0, 1))
        return jnp.where(keep, x, tv)

    u = fix(u_ref[:, :32], uid_ref[...], tu_ref, utailo)
    i = fix(i_ref[:, :32], iid_ref[...], ti_ref, itailo)
    h = (
        jnp.dot(u, w1u_ref[...], preferred_element_type=jnp.float32)
        + jnp.dot(i, w1i_ref[...], preferred_element_type=jnp.float32)
        + b1_ref[...]
    )
    h = jnp.maximum(h, 0.0)
    h = jnp.dot(h, w2_ref[...], preferred_element_type=jnp.float32) + b2_ref[...]
    h = jnp.maximum(h, 0.0)
    z = jnp.sum(h * w3_ref[...], axis=1) + b3_ref[...]
    o_ref[...] = 1.0 / (1.0 + jnp.exp(-z))


def _tc_mlp(u, i, uid, iid, tail_u, tail_i, W1, b1, W2, b2, W3, b3):
    B = u.shape[0]
    D = 32
    BT = 8192
    w1u = W1[:, :D].T
    w1i = W1[:, D:].T
    w2 = W2.T
    w3 = W3[0]
    grid = (B // BT,)
    body = functools.partial(_mlp_body, NUM_USERS_TAILO, NUM_ITEMS_TAILO)
    return pl.pallas_call(
        body,
        grid=grid,
        in_specs=[
            pl.BlockSpec((BT, 128), lambda g: (g, 0)),
            pl.BlockSpec((BT, 128), lambda g: (g, 0)),
            pl.BlockSpec((BT, 1), lambda g: (g, 0)),
            pl.BlockSpec((BT, 1), lambda g: (g, 0)),
            pl.BlockSpec(tail_u.shape, lambda g: (0, 0)),
            pl.BlockSpec(tail_i.shape, lambda g: (0, 0)),
            pl.BlockSpec(w1u.shape, lambda g: (0, 0)),
            pl.BlockSpec(w1i.shape, lambda g: (0, 0)),
            pl.BlockSpec(b1.shape, lambda g: (0,)),
            pl.BlockSpec(w2.shape, lambda g: (0, 0)),
            pl.BlockSpec(b2.shape, lambda g: (0,)),
            pl.BlockSpec(w3.shape, lambda g: (0,)),
            pl.BlockSpec(b3.shape, lambda g: (0,)),
        ],
        out_specs=pl.BlockSpec((BT,), lambda g: (g,)),
        out_shape=jax.ShapeDtypeStruct((B,), jnp.float32),
    )(u, i, uid.reshape(B, 1), iid.reshape(B, 1), tail_u, tail_i, w1u, w1i, b1, w2, b2, w3, b3)


NUM_USERS_TAILO = 999936
NUM_ITEMS_TAILO = 99968


def kernel(user_ids, item_ids, user_table, item_table, W1, b1, W2, b2, W3, b3):
    B = user_ids.shape[0]
    uid = user_ids.astype(jnp.int32)
    iid = item_ids.astype(jnp.int32)
    pos = lax.iota(jnp.int32, B)
    su, pu = lax.sort((uid, pos), num_keys=1)
    si, pi = lax.sort((iid, pos), num_keys=1)
    _, inv_pu = lax.sort((pu, pos), num_keys=1)
    _, inv_pi = lax.sort((pi, pos), num_keys=1)
    semb_u, semb_i = _sc_scan(su, si, user_table.T, item_table.T)
    u, i = _sc_unpermute(semb_u, semb_i, inv_pu, inv_pi)
    tail_u = user_table[NUM_USERS_TAILO:, :]
    tail_i = item_table[NUM_ITEMS_TAILO:, :]
    return _tc_mlp(u, i, uid, iid, tail_u, tail_i, W1, b1, W2, b2, W3, b3)


# speedup vs baseline: 1.0125x; 1.0125x over previous
import functools

import jax
import jax.numpy as jnp
from jax import lax
from jax.experimental import pallas as pl
from jax.experimental.pallas import tpu as pltpu
from jax.experimental.pallas import tpu_sc as plsc

NC = 2
NS = 16
NW = NC * NS
L = 16
WIN = 512
NF = 32


def _scan_table(tabT, sids, gbmin, gbmax, cbuf, wbuf0, wbuf1, wbuf2,
                sem0, sem1, sem2, out, base, bpw):
    n_users = tabT.shape[1]
    lasta = ((n_users - WIN) // 128) * 128
    ngrp = bpw // L

    i16 = lax.iota(jnp.int32, L)
    for h in range(ngrp // L):
        gbmin[pl.ds(h * L, L)] = plsc.load_gather(
            sids, [i16 * L + (h * L * L)])
        gbmax[pl.ds(h * L, L)] = plsc.load_gather(
            sids, [i16 * L + (h * L * L + L - 1)])

    first = sids[pl.ds(0, L)][0]
    last = sids[pl.ds(bpw - L, L)][L - 1]
    wlo0 = jnp.minimum((first >> 7) << 7, lasta)
    nwin = (((last >> 7) << 7) - wlo0) // WIN + 1
    nwin3 = ((nwin + 2) // 3) * 3

    def wstart(k):
        w = jnp.minimum(wlo0 + k * WIN, lasta)
        return pl.multiple_of(w, 128)

    def fire(k, buf, sem):
        pltpu.async_copy(tabT.at[:, pl.ds(wstart(k), WIN)], buf, sem)

    def drain(buf, sem):
        pltpu.make_async_copy(tabT.at[:, pl.ds(0, WIN)], buf, sem).wait()

    def extract(k, buf):
        wlo = wstart(k)
        whi = wlo + WIN
        gl = jnp.int32(0)
        gh = jnp.int32(0)
        for h in range(ngrp // L):
            mx = gbmax[pl.ds(h * L, L)]
            mn = gbmin[pl.ds(h * L, L)]
            gl = gl + plsc.all_reduce_population_count(mx < wlo)[0]
            gh = gh + plsc.all_reduce_population_count(mn < whi)[0]

        @pl.loop(gl, gh)
        def _(g):
            gids = sids[pl.ds(g * L, L)]
            rel = gids - wlo
            m = jnp.logical_and(gids >= wlo, gids < whi)
            rows = i16 + g * L
            for f in range(NF):
                fvec = jnp.full((L,), f, jnp.int32)
                vals = plsc.load_gather(buf, [fvec, rel], mask=m)
                plsc.store_scatter(cbuf, [rows, fvec], vals, mask=m)

    fire(0, wbuf0, sem0)
    fire(1, wbuf1, sem1)
    fire(2, wbuf2, sem2)

    @pl.loop(0, nwin3, step=3)
    def _(j):
        drain(wbuf0, sem0)
        extract(j, wbuf0)
        fire(j + 3, wbuf0, sem0)
        drain(wbuf1, sem1)
        extract(j + 1, wbuf1)
        fire(j + 4, wbuf1, sem1)
        drain(wbuf2, sem2)
        extract(j + 2, wbuf2)
        fire(j + 5, wbuf2, sem2)

    drain(wbuf0, sem0)
    drain(wbuf1, sem1)
    drain(wbuf2, sem2)

    pltpu.sync_copy(cbuf, out.at[pl.ds(base, bpw)])


def _scan_body(bpw, su, si, utabT, itabT, uout, iout,
               sids, gbmin, gbmax, cbuf, wbuf0, wbuf1, wbuf2,
               sem0, sem1, sem2):
    wid = lax.axis_index("s") * NC + lax.axis_index("c")
    base = wid * bpw
    pltpu.sync_copy(su.at[pl.ds(base, bpw)], sids)
    _scan_table(utabT, sids, gbmin, gbmax, cbuf, wbuf0, wbuf1, wbuf2,
                sem0, sem1, sem2, uout, base, bpw)
    pltpu.sync_copy(si.at[pl.ds(base, bpw)], sids)
    _scan_table(itabT, sids, gbmin, gbmax, cbuf, wbuf0, wbuf1, wbuf2,
                sem0, sem1, sem2, iout, base, bpw)


def _sc_scan(su, si, utabT, itabT):
    B = su.shape[0]
    bpw = B // NW
    body = functools.partial(_scan_body, bpw)
    out2 = jax.ShapeDtypeStruct((B, 128), jnp.float32)
    mesh = plsc.VectorSubcoreMesh(
        core_axis_name="c", subcore_axis_name="s", num_cores=NC, num_subcores=NS
    )
    k = pl.kernel(
        body,
        out_type=(out2, out2),
        mesh=mesh,
        compiler_params=pltpu.CompilerParams(needs_layout_passes=False),
        scratch_types=[
            pltpu.VMEM((bpw,), jnp.int32),
            pltpu.VMEM((bpw // L,), jnp.int32),
            pltpu.VMEM((bpw // L,), jnp.int32),
            pltpu.VMEM((bpw, 128), jnp.float32),
            pltpu.VMEM((NF, WIN), jnp.float32),
            pltpu.VMEM((NF, WIN), jnp.float32),
            pltpu.VMEM((NF, WIN), jnp.float32),
            pltpu.SemaphoreType.DMA,
            pltpu.SemaphoreType.DMA,
            pltpu.SemaphoreType.DMA,
        ],
    )
    return k(su, si, utabT, itabT)


def _unperm_body(bpw, semu, semi, invu, invi, uout, iout,
                 uidx, iidx, ub0, ub1, ib0, ib1, sem0, sem1):
    wid = lax.axis_index("s") * NC + lax.axis_index("c")
    nchunk = bpw // 128
    pltpu.sync_copy(invu.at[pl.ds(wid * nchunk, nchunk)], uidx)
    pltpu.sync_copy(invi.at[pl.ds(wid * nchunk, nchunk)], iidx)

    def fire(j, ub, ib, sem):
        pltpu.async_copy(semu.at[uidx.at[j]], ub, sem)
        pltpu.async_copy(semi.at[iidx.at[j]], ib, sem)

    def drain(ub, ib, sem):
        pltpu.make_async_copy(semu.at[uidx.at[0]], ub, sem).wait()
        pltpu.make_async_copy(semi.at[iidx.at[0]], ib, sem).wait()

    fire(0, ub0, ib0, sem0)
    fire(1, ub1, ib1, sem1)
    for j in range(nchunk):
        ub, ib, sem = (ub0, ib0, sem0) if j % 2 == 0 else (ub1, ib1, sem1)
        drain(ub, ib, sem)
        row = wid * bpw + j * 128
        pltpu.sync_copy(ub, uout.at[pl.ds(row, 128)])
        pltpu.sync_copy(ib, iout.at[pl.ds(row, 128)])
        if j + 2 < nchunk:
            fire(j + 2, ub, ib, sem)


def _sc_unpermute(semb_u, semb_i, inv_pu, inv_pi):
    B = semb_u.shape[0]
    bpw = B // NW
    nchunk = bpw // 128
    invu2 = inv_pu.reshape(B // 128, 128)
    invi2 = inv_pi.reshape(B // 128, 128)
    body = functools.partial(_unperm_body, bpw)
    out2 = jax.ShapeDtypeStruct((B, 128), jnp.float32)
    mesh = plsc.VectorSubcoreMesh(
        core_axis_name="c", subcore_axis_name="s", num_cores=NC, num_subcores=NS
    )
    k = pl.kernel(
        body,
        out_type=(out2, out2),
        mesh=mesh,
        scratch_types=[
            pltpu.VMEM((nchunk, 128), jnp.int32),
            pltpu.VMEM((nchunk, 128), jnp.int32),
            pltpu.VMEM((128, 128), jnp.float32),
            pltpu.VMEM((128, 128), jnp.float32),
            pltpu.VMEM((128, 128), jnp.float32),
            pltpu.VMEM((128, 128), jnp.float32),
            pltpu.SemaphoreType.DMA,
            pltpu.SemaphoreType.DMA,
        ],
    )
    return k(semb_u, semb_i, invu2, invi2)


def _mlp_body(utailo, itailo, u_ref, i_ref, uid_ref, iid_ref, tu_ref, ti_ref,
              w1u_ref, w1i_ref, b1_ref, w2_ref, b2_ref,
              w3_ref, b3_ref, o_ref):
    bt = u_ref.shape[0]

    def fix(x, ids2, tail_ref, tailo):
        n = tail_ref.shape[0]
        idb = lax.broadcast_in_dim(ids2, (bt, n), (0, 1))
        rel = jnp.clip(idb - tailo, 0, n - 1)
        oh = (rel == lax.broadcasted_iota(jnp.int32, (bt, n), 1))
        tv = jnp.dot(oh.astype(jnp.float32), tail_ref[...],
                     preferred_element_type=jnp.float32)
        keep = lax.broadcast_in_dim(ids2 < tailo, (bt, x.shape[1]), (0, 1))
        return jnp.where(keep, x, tv)

    u = fix(u_ref[:, :32], uid_ref[...], tu_ref, utailo)
    i = fix(i_ref[:, :32], iid_ref[...], ti_ref, itailo)
    h = (
        jnp.dot(u, w1u_ref[...], preferred_element_type=jnp.float32)
        + jnp.dot(i, w1i_ref[...], preferred_element_type=jnp.float32)
        + b1_ref[...]
    )
    h = jnp.maximum(h, 0.0)
    h = jnp.dot(h, w2_ref[...], preferred_element_type=jnp.float32) + b2_ref[...]
    h = jnp.maximum(h, 0.0)
    z = jnp.sum(h * w3_ref[...], axis=1) + b3_ref[...]
    o_ref[...] = 1.0 / (1.0 + jnp.exp(-z))


def _tc_mlp(u, i, uid, iid, tail_u, tail_i, W1, b1, W2, b2, W3, b3):
    B = u.shape[0]
    D = 32
    BT = 4096
    w1u = W1[:, :D].T
    w1i = W1[:, D:].T
    w2 = W2.T
    w3 = W3[0]
    grid = (B // BT,)
    body = functools.partial(_mlp_body, NUM_USERS_TAILO, NUM_ITEMS_TAILO)
    return pl.pallas_call(
        body,
        grid=grid,
        in_specs=[
            pl.BlockSpec((BT, 128), lambda g: (g, 0)),
            pl.BlockSpec((BT, 128), lambda g: (g, 0)),
            pl.BlockSpec((BT, 1), lambda g: (g, 0)),
            pl.BlockSpec((BT, 1), lambda g: (g, 0)),
            pl.BlockSpec(tail_u.shape, lambda g: (0, 0)),
            pl.BlockSpec(tail_i.shape, lambda g: (0, 0)),
            pl.BlockSpec(w1u.shape, lambda g: (0, 0)),
            pl.BlockSpec(w1i.shape, lambda g: (0, 0)),
            pl.BlockSpec(b1.shape, lambda g: (0,)),
            pl.BlockSpec(w2.shape, lambda g: (0, 0)),
            pl.BlockSpec(b2.shape, lambda g: (0,)),
            pl.BlockSpec(w3.shape, lambda g: (0,)),
            pl.BlockSpec(b3.shape, lambda g: (0,)),
        ],
        out_specs=pl.BlockSpec((BT,), lambda g: (g,)),
        out_shape=jax.ShapeDtypeStruct((B,), jnp.float32),
    )(u, i, uid.reshape(B, 1), iid.reshape(B, 1), tail_u, tail_i, w1u, w1i, b1, w2, b2, w3, b3)


NUM_USERS_TAILO = 999936
NUM_ITEMS_TAILO = 99968


def kernel(user_ids, item_ids, user_table, item_table, W1, b1, W2, b2, W3, b3):
    B = user_ids.shape[0]
    uid = user_ids.astype(jnp.int32)
    iid = item_ids.astype(jnp.int32)
    pos = lax.iota(jnp.int32, B)
    su, pu = lax.sort((uid, pos), num_keys=1)
    si, pi = lax.sort((iid, pos), num_keys=1)
    _, inv_pu = lax.sort((pu, pos), num_keys=1)
    _, inv_pi = lax.sort((pi, pos), num_keys=1)
    semb_u, semb_i = _sc_scan(su, si, user_table.T, item_table.T)
    u, i = _sc_unpermute(semb_u, semb_i, inv_pu, inv_pi)
    tail_u = user_table[NUM_USERS_TAILO:, :]
    tail_i = item_table[NUM_ITEMS_TAILO:, :]
    return _tc_mlp(u, i, uid, iid, tail_u, tail_i, W1, b1, W2, b2, W3, b3)
